# baseline (speedup 1.0000x reference)
import functools

import jax
import jax.numpy as jnp
from jax import lax
from jax.experimental import pallas as pl
from jax.experimental.pallas import tpu as pltpu
from jax.experimental.pallas import tpu_sc as plsc

_NUM_CORES = 2
_NUM_SUBCORES = 16
_NUM_WORKERS = _NUM_CORES * _NUM_SUBCORES

_GROUP = 4

_ROW = 128


def _gather_kernel(batch, length, emb, table_hbm, idx_hbm, out_hbm,
                   idx_v, buf0, buf1, sem0, sem1):
    rows_w = batch // _NUM_WORKERS
    n_groups = rows_w // _GROUP
    wid = lax.axis_index("s") * _NUM_CORES + lax.axis_index("c")
    row_base = wid * rows_w

    pltpu.sync_copy(idx_hbm.at[pl.ds(row_base, rows_w)], idx_v)

    def fire(buf, sem, g):
        for j in range(_GROUP):
            pltpu.async_copy(
                table_hbm.at[idx_v.at[g * _GROUP + j]],
                buf.at[j],
                sem,
            )

    def drain(buf, sem):
        pltpu.make_async_copy(out_hbm.at[pl.ds(0, _GROUP)], buf, sem).wait()

    def writeback(buf, g):
        pltpu.sync_copy(
            buf, out_hbm.at[pl.ds(row_base + g * _GROUP, _GROUP)])

    fire(buf0, sem0, 0)
    fire(buf1, sem1, 1)

    def body(t, carry):
        g0 = 2 * t
        g1 = g0 + 1

        drain(buf0, sem0)
        writeback(buf0, g0)

        @pl.when(g0 + 2 < n_groups)
        def _():
            fire(buf0, sem0, g0 + 2)

        drain(buf1, sem1)
        writeback(buf1, g1)

        @pl.when(g1 + 2 < n_groups)
        def _():
            fire(buf1, sem1, g1 + 2)

        return carry

    lax.fori_loop(0, n_groups // 2, body, 0)


_VREG = 16
_CHUNK = 512


def _pad_kernel(n_rows, n_pad, emb, table_hbm, tail_hbm, pad_hbm, in_v, out_v):
    wid = lax.axis_index("s") * _NUM_CORES + lax.axis_index("c")
    per_w = n_rows // _NUM_WORKERS // 8 * 8
    n_chunks = per_w // _CHUNK
    tail = per_w - n_chunks * _CHUNK
    row0 = wid * per_w

    def expand(base, count):
        pltpu.sync_copy(table_hbm.at[pl.ds(base, count)],
                        in_v.at[pl.ds(0, count)])
        for r in range(count):
            for c in range(emb // _VREG):
                out_v[r, pl.ds(c * _VREG, _VREG)] = (
                    in_v[r, pl.ds(c * _VREG, _VREG)])
        pltpu.sync_copy(out_v.at[pl.ds(0, count)],
                        pad_hbm.at[pl.ds(base, count)])

    def body(k, carry):
        expand(row0 + k * _CHUNK, _CHUNK)
        return carry

    lax.fori_loop(0, n_chunks, body, 0)
    if tail:
        expand(row0 + n_chunks * _CHUNK, tail)

    rem8 = (n_rows - per_w * _NUM_WORKERS) // 8 * 8
    if rem8:
        @pl.when(wid == 0)
        def _():
            expand(per_w * _NUM_WORKERS, rem8)
    if per_w * _NUM_WORKERS + rem8 < n_rows:
        @pl.when(wid == 0)
        def _():
            pltpu.sync_copy(tail_hbm, in_v.at[pl.ds(0, 8)])
            for c in range(emb // _VREG):
                out_v[0, pl.ds(c * _VREG, _VREG)] = (
                    in_v[0, pl.ds(c * _VREG, _VREG)])
            pltpu.sync_copy(out_v.at[pl.ds(0, 8)],
                            pad_hbm.at[pl.ds(n_rows - 1, 8)])


def kernel(indices, table):
    batch, length = indices.shape
    emb = table.shape[1]
    rows_w = batch // _NUM_WORKERS

    idx = indices.astype(jnp.int32)

    n_rows = table.shape[0]
    n_pad = (n_rows + _NUM_WORKERS * 8 - 1) // (_NUM_WORKERS * 8) * (
        _NUM_WORKERS * 8)
    mesh = plsc.VectorSubcoreMesh(core_axis_name="c", subcore_axis_name="s")
    tail = jnp.tile(table[n_rows - 1:n_rows], (8, 1))
    table_pad = pl.kernel(
        functools.partial(_pad_kernel, n_rows, n_pad, emb),
        mesh=mesh,
        out_type=jax.ShapeDtypeStruct((n_pad, _ROW), jnp.float32),
        scratch_types=[
            pltpu.VMEM((_CHUNK, emb), jnp.float32),
            pltpu.VMEM((_CHUNK, _ROW), jnp.float32),
        ],
        compiler_params=pltpu.CompilerParams(use_tc_tiling_on_sc=True),
    )(table, tail)

    out = pl.kernel(
        functools.partial(_gather_kernel, batch, length, emb),
        mesh=mesh,
        out_type=jax.ShapeDtypeStruct((batch, length, _ROW), jnp.float32),
        scratch_types=[
            pltpu.VMEM((rows_w, length), jnp.int32),
            pltpu.VMEM((_GROUP, length, _ROW), jnp.float32),
            pltpu.VMEM((_GROUP, length, _ROW), jnp.float32),
            pltpu.SemaphoreType.DMA,
            pltpu.SemaphoreType.DMA,
        ],
        compiler_params=pltpu.CompilerParams(use_tc_tiling_on_sc=True),
    )(table_pad, idx)
    return out[..., :emb]
